# baseline (speedup 1.0000x reference)
import jax
import jax.numpy as jnp
from jax.experimental import pallas as pl
from jax.experimental.pallas import tpu as pltpu

_B = 4
_N = 2048
_E = 16
_CAP = 160
_NBLK = 256
_NB = _N // _NBLK


def _zero_kernel(p_ref, comb_ref, disp_ref):
    v = p_ref[0, 0]
    comb_ref[...] = jnp.full(comb_ref.shape, 0.0, jnp.float32) + v
    disp_ref[...] = jnp.full(disp_ref.shape, 0.0, jnp.float32) + v


def kernel(x, w_gating, probs):
    comb, disp = pl.pallas_call(
        _zero_kernel,
        grid=(_B, _NB),
        in_specs=[pl.BlockSpec((1, 1), lambda b, nb: (0, 0),
                               memory_space=pltpu.SMEM)],
        out_specs=[
            pl.BlockSpec((1, _NBLK, _E, _CAP), lambda b, nb: (b, nb, 0, 0)),
            pl.BlockSpec((1, _NBLK, _E, _CAP), lambda b, nb: (b, nb, 0, 0)),
        ],
        out_shape=[
            jax.ShapeDtypeStruct((_B, _N, _E, _CAP), jnp.float32),
            jax.ShapeDtypeStruct((_B, _N, _E, _CAP), jnp.float32),
        ],
    )(probs[:1, :1])
    return (disp, comb, jnp.float32(0.0), jnp.float32(0.0))


# speedup vs baseline: 3.0124x; 3.0124x over previous
import jax
import jax.numpy as jnp
from jax.experimental import pallas as pl
from jax.experimental.pallas import tpu as pltpu

_B = 4
_N = 2048
_DIM = 4096
_E = 16
_CAP = 160
_EPS = 1e-9
_THRESH = 0.2
_NBLK = 256
_NB = _N // _NBLK


def _gating_kernel(x_ref, w_ref, p_ref, comb_ref, disp_ref, bal_ref, z_ref,
                   c1_ref, c2_ref, sp_ref, sd_ref):
    b = pl.program_id(0)
    nb = pl.program_id(1)

    @pl.when(jnp.logical_and(b == 0, nb == 0))
    def _init_outs():
        bal_ref[...] = jnp.zeros_like(bal_ref)
        z_ref[...] = jnp.zeros_like(z_ref)

    @pl.when(nb == 0)
    def _init_carries():
        c1_ref[...] = jnp.zeros_like(c1_ref)
        c2_ref[...] = jnp.zeros_like(c2_ref)
        sp_ref[...] = jnp.zeros_like(sp_ref)
        sd_ref[...] = jnp.zeros_like(sd_ref)

    x = x_ref[0]
    w = w_ref[...]
    logits_t = jax.lax.dot_general(
        w, x, (((0,), (1,)), ((), ())),
        preferred_element_type=jnp.float32)

    m = jnp.max(logits_t, axis=0, keepdims=True)
    ex = jnp.exp(logits_t - m)
    s = jnp.sum(ex, axis=0, keepdims=True)
    gates = ex / s
    lse = m + jnp.log(s)
    z_ref[...] += jnp.sum(lse * lse, axis=(0, 1), keepdims=True)

    iota_e = jax.lax.broadcasted_iota(jnp.int32, (_E, _NBLK), 0)
    g1 = jnp.max(gates, axis=0, keepdims=True)
    i1 = jnp.min(jnp.where(gates == g1, iota_e, _E), axis=0, keepdims=True)
    gm = jnp.where(iota_e == i1, -1.0, gates)
    g2 = jnp.max(gm, axis=0, keepdims=True)
    i2 = jnp.min(jnp.where(gm == g2, iota_e, _E), axis=0, keepdims=True)

    m1 = (iota_e == i1).astype(jnp.float32)
    denom = g1 + g2 + _EPS
    g1n = g1 / denom
    g2n = g2 / denom

    probs = p_ref[0]
    route2 = probs < (g2n / _THRESH)
    m2 = (iota_e == i2).astype(jnp.float32) * route2.astype(jnp.float32)

    r_i = jax.lax.broadcasted_iota(jnp.int32, (_NBLK, _NBLK), 0)
    c_i = jax.lax.broadcasted_iota(jnp.int32, (_NBLK, _NBLK), 1)
    s_upper = (r_i < c_i).astype(jnp.float32)
    excl1 = jnp.dot(m1, s_upper, preferred_element_type=jnp.float32)
    excl2 = jnp.dot(m2, s_upper, preferred_element_type=jnp.float32)

    pos1 = jnp.sum((excl1 + c1_ref[...]) * m1, axis=0, keepdims=True)
    pos2 = jnp.sum((excl2 + c2_ref[...]) * m2, axis=0, keepdims=True)

    keep1 = (pos1 < float(_CAP)).astype(jnp.float32)
    g1f = g1n * keep1
    routed2 = jnp.sum(m2, axis=0, keepdims=True) > 0.0
    keep2 = jnp.where(routed2, 1.0, 0.0) * (pos2 < float(_CAP)).astype(
        jnp.float32)
    g2f = g2n * keep2

    c1_ref[...] += jnp.sum(m1, axis=1, keepdims=True)
    c2_ref[...] += jnp.sum(m2, axis=1, keepdims=True)
    sp_ref[...] += jnp.sum(gates, axis=1, keepdims=True)
    sd_ref[...] += jnp.sum(m1, axis=1, keepdims=True)

    @pl.when(nb == _NB - 1)
    def _fold_balance():
        bal_ref[...] += jnp.sum(sp_ref[...] * sd_ref[...], axis=(0, 1),
                                keepdims=True)

    p1i = pos1.astype(jnp.int32)
    p2i = pos2.astype(jnp.int32)
    c_iota = jax.lax.broadcasted_iota(jnp.int32, (_CAP, _NBLK), 0)
    for e in range(_E):
        key1 = jnp.where(i1 == e, p1i, -1)
        key2 = jnp.where(jnp.logical_and(i2 == e, routed2), p2i, -1)
        eq1 = c_iota == key1
        eq2 = c_iota == key2
        comb = jnp.where(eq2, g2f, jnp.where(eq1, g1f, 0.0))
        disp = jnp.where(jnp.logical_or(eq1, eq2), 1.0, 0.0)
        comb_ref[0, e * _CAP:(e + 1) * _CAP, :] = comb
        disp_ref[0, e * _CAP:(e + 1) * _CAP, :] = disp


def _run_gating(x, w_gating, probs3, interpret=False):
    return pl.pallas_call(
        _gating_kernel,
        grid=(_B, _NB),
        in_specs=[
            pl.BlockSpec((1, _NBLK, _DIM), lambda b, nb: (b, nb, 0)),
            pl.BlockSpec((_DIM, _E), lambda b, nb: (0, 0)),
            pl.BlockSpec((1, 1, _NBLK), lambda b, nb: (b, 0, nb)),
        ],
        out_specs=[
            pl.BlockSpec((1, _E * _CAP, _NBLK), lambda b, nb: (b, 0, nb)),
            pl.BlockSpec((1, _E * _CAP, _NBLK), lambda b, nb: (b, 0, nb)),
            pl.BlockSpec((1, 1), lambda b, nb: (0, 0)),
            pl.BlockSpec((1, 1), lambda b, nb: (0, 0)),
        ],
        out_shape=[
            jax.ShapeDtypeStruct((_B, _E * _CAP, _N), jnp.float32),
            jax.ShapeDtypeStruct((_B, _E * _CAP, _N), jnp.float32),
            jax.ShapeDtypeStruct((1, 1), jnp.float32),
            jax.ShapeDtypeStruct((1, 1), jnp.float32),
        ],
        scratch_shapes=[
            pltpu.VMEM((_E, 1), jnp.float32),
            pltpu.VMEM((_E, 1), jnp.float32),
            pltpu.VMEM((_E, 1), jnp.float32),
            pltpu.VMEM((_E, 1), jnp.float32),
        ],
        interpret=interpret,
    )(x, w_gating, probs3)


def kernel(x, w_gating, probs):
    probs3 = probs.reshape(_B, 1, _N)
    comb_t, disp_t, bal, z = _run_gating(x, w_gating, probs3)
    combine_tensor = comb_t.reshape(_B, _E, _CAP, _N).transpose(0, 3, 1, 2)
    dispatch_tensor = disp_t.reshape(_B, _E, _CAP, _N).transpose(0, 3, 1, 2)
    balance_loss = bal[0, 0] * (float(_E * _E) / float(_B * _E * _N * _N))
    router_z_loss = z[0, 0] / float(_B * _N)
    return (dispatch_tensor, combine_tensor, balance_loss, router_z_loss)


# speedup vs baseline: 3.0641x; 1.0171x over previous
import jax
import jax.numpy as jnp
from jax.experimental import pallas as pl
from jax.experimental.pallas import tpu as pltpu

_B = 4
_N = 2048
_DIM = 4096
_E = 16
_CAP = 160
_EPS = 1e-9
_THRESH = 0.2
_NBLK = 512
_NB = _N // _NBLK


def _gating_kernel(x_ref, w_ref, p_ref, comb_ref, disp_ref, bal_ref, z_ref,
                   c1_ref, c2_ref, sp_ref, sd_ref):
    b = pl.program_id(0)
    nb = pl.program_id(1)

    @pl.when(jnp.logical_and(b == 0, nb == 0))
    def _init_outs():
        bal_ref[...] = jnp.zeros_like(bal_ref)
        z_ref[...] = jnp.zeros_like(z_ref)

    @pl.when(nb == 0)
    def _init_carries():
        c1_ref[...] = jnp.zeros_like(c1_ref)
        c2_ref[...] = jnp.zeros_like(c2_ref)
        sp_ref[...] = jnp.zeros_like(sp_ref)
        sd_ref[...] = jnp.zeros_like(sd_ref)

    x = x_ref[0]
    w = w_ref[...]
    logits_t = jax.lax.dot_general(
        w, x, (((0,), (1,)), ((), ())),
        preferred_element_type=jnp.float32)

    m = jnp.max(logits_t, axis=0, keepdims=True)
    ex = jnp.exp(logits_t - m)
    s = jnp.sum(ex, axis=0, keepdims=True)
    gates = ex / s
    lse = m + jnp.log(s)
    z_ref[...] += jnp.sum(lse * lse, axis=(0, 1), keepdims=True)

    iota_e = jax.lax.broadcasted_iota(jnp.int32, (_E, _NBLK), 0)
    g1 = jnp.max(gates, axis=0, keepdims=True)
    i1 = jnp.min(jnp.where(gates == g1, iota_e, _E), axis=0, keepdims=True)
    gm = jnp.where(iota_e == i1, -1.0, gates)
    g2 = jnp.max(gm, axis=0, keepdims=True)
    i2 = jnp.min(jnp.where(gm == g2, iota_e, _E), axis=0, keepdims=True)

    m1 = (iota_e == i1).astype(jnp.float32)
    denom = g1 + g2 + _EPS
    g1n = g1 / denom
    g2n = g2 / denom

    probs = p_ref[0]
    route2 = probs < (g2n / _THRESH)
    m2 = (iota_e == i2).astype(jnp.float32) * route2.astype(jnp.float32)

    r_i = jax.lax.broadcasted_iota(jnp.int32, (_NBLK, _NBLK), 0)
    c_i = jax.lax.broadcasted_iota(jnp.int32, (_NBLK, _NBLK), 1)
    s_upper = (r_i < c_i).astype(jnp.float32)
    excl1 = jnp.dot(m1, s_upper, preferred_element_type=jnp.float32)
    excl2 = jnp.dot(m2, s_upper, preferred_element_type=jnp.float32)

    pos1 = jnp.sum((excl1 + c1_ref[...]) * m1, axis=0, keepdims=True)
    pos2 = jnp.sum((excl2 + c2_ref[...]) * m2, axis=0, keepdims=True)

    keep1 = (pos1 < float(_CAP)).astype(jnp.float32)
    g1f = g1n * keep1
    routed2 = jnp.sum(m2, axis=0, keepdims=True) > 0.0
    keep2 = jnp.where(routed2, 1.0, 0.0) * (pos2 < float(_CAP)).astype(
        jnp.float32)
    g2f = g2n * keep2

    c1_ref[...] += jnp.sum(m1, axis=1, keepdims=True)
    c2_ref[...] += jnp.sum(m2, axis=1, keepdims=True)
    sp_ref[...] += jnp.sum(gates, axis=1, keepdims=True)
    sd_ref[...] += jnp.sum(m1, axis=1, keepdims=True)

    @pl.when(nb == _NB - 1)
    def _fold_balance():
        bal_ref[...] += jnp.sum(sp_ref[...] * sd_ref[...], axis=(0, 1),
                                keepdims=True)

    p1i = pos1.astype(jnp.int32)
    p2i = pos2.astype(jnp.int32)
    c_iota = jax.lax.broadcasted_iota(jnp.int32, (_CAP, _NBLK), 0)
    for e in range(_E):
        key1 = jnp.where(i1 == e, p1i, -1)
        key2 = jnp.where(jnp.logical_and(i2 == e, routed2), p2i, -1)
        eq1 = c_iota == key1
        eq2 = c_iota == key2
        comb = jnp.where(eq2, g2f, jnp.where(eq1, g1f, 0.0))
        disp = jnp.where(jnp.logical_or(eq1, eq2), 1.0, 0.0)
        comb_ref[0, e * _CAP:(e + 1) * _CAP, :] = comb
        disp_ref[0, e * _CAP:(e + 1) * _CAP, :] = disp


def _run_gating(x, w_gating, probs3, interpret=False):
    return pl.pallas_call(
        _gating_kernel,
        grid=(_B, _NB),
        in_specs=[
            pl.BlockSpec((1, _NBLK, _DIM), lambda b, nb: (b, nb, 0)),
            pl.BlockSpec((_DIM, _E), lambda b, nb: (0, 0)),
            pl.BlockSpec((1, 1, _NBLK), lambda b, nb: (b, 0, nb)),
        ],
        out_specs=[
            pl.BlockSpec((1, _E * _CAP, _NBLK), lambda b, nb: (b, 0, nb)),
            pl.BlockSpec((1, _E * _CAP, _NBLK), lambda b, nb: (b, 0, nb)),
            pl.BlockSpec((1, 1), lambda b, nb: (0, 0)),
            pl.BlockSpec((1, 1), lambda b, nb: (0, 0)),
        ],
        out_shape=[
            jax.ShapeDtypeStruct((_B, _E * _CAP, _N), jnp.float32),
            jax.ShapeDtypeStruct((_B, _E * _CAP, _N), jnp.float32),
            jax.ShapeDtypeStruct((1, 1), jnp.float32),
            jax.ShapeDtypeStruct((1, 1), jnp.float32),
        ],
        scratch_shapes=[
            pltpu.VMEM((_E, 1), jnp.float32),
            pltpu.VMEM((_E, 1), jnp.float32),
            pltpu.VMEM((_E, 1), jnp.float32),
            pltpu.VMEM((_E, 1), jnp.float32),
        ],
        interpret=interpret,
    )(x, w_gating, probs3)


def kernel(x, w_gating, probs):
    probs3 = probs.reshape(_B, 1, _N)
    comb_t, disp_t, bal, z = _run_gating(x, w_gating, probs3)
    combine_tensor = comb_t.reshape(_B, _E, _CAP, _N).transpose(0, 3, 1, 2)
    dispatch_tensor = disp_t.reshape(_B, _E, _CAP, _N).transpose(0, 3, 1, 2)
    balance_loss = bal[0, 0] * (float(_E * _E) / float(_B * _E * _N * _N))
    router_z_loss = z[0, 0] / float(_B * _N)
    return (dispatch_tensor, combine_tensor, balance_loss, router_z_loss)
